# baseline (speedup 1.0000x reference)
import functools

import jax
import jax.numpy as jnp
from jax import lax
from jax.experimental import pallas as pl
from jax.experimental.pallas import tpu as pltpu
from jax.experimental.pallas import tpu_sc as plsc

_E = 8
_AUX_W = 0.01
_TM = 256
_RMAX = 4096 + _E * _TM
_NTILES = _RMAX // _TM


def _wq_body(w_ref, q_ref):
    w = w_ref[0]
    s = jnp.maximum(jnp.mean(jnp.abs(w)), 1e-8)
    uq = jnp.clip(jnp.round(w / s), -1.0, 1.0)
    q_ref[0] = (uq * s).astype(jnp.bfloat16)


def _quantize_weights(w):
    e, r, c = w.shape
    return pl.pallas_call(
        _wq_body,
        grid=(e,),
        in_specs=[pl.BlockSpec((1, r, c), lambda i: (i, 0, 0))],
        out_specs=pl.BlockSpec((1, r, c), lambda i: (i, 0, 0)),
        out_shape=jax.ShapeDtypeStruct((e, r, c), jnp.bfloat16),
    )(w)


def _router_body(x_ref, wr_ref, xq_ref, p1_ref, p2_ref, w1_ref, w2_ref,
                 tem_ref, aux_ref):
    n = x_ref.shape[0]
    x = x_ref[...]
    logits = lax.dot_general(x.astype(jnp.bfloat16),
                             wr_ref[...].astype(jnp.bfloat16),
                             (((1,), (1,)), ((), ())),
                             preferred_element_type=jnp.float32)
    mx = jnp.max(logits, axis=1, keepdims=True)
    ex = jnp.exp(logits - mx)
    probs = ex / jnp.sum(ex, axis=1, keepdims=True)

    e = probs.shape[1]
    iota = lax.broadcasted_iota(jnp.int32, (n, e), 1)
    m1 = jnp.max(probs, axis=1, keepdims=True)
    i1 = jnp.min(jnp.where(probs == m1, iota, e), axis=1, keepdims=True)
    sel1 = iota == i1
    pm = jnp.where(sel1, -1.0, probs)
    m2 = jnp.max(pm, axis=1, keepdims=True)
    i2 = jnp.min(jnp.where(pm == m2, iota, e), axis=1, keepdims=True)
    sel2 = iota == i2
    denom = jnp.maximum(m1 + m2, 1e-9)
    w1_ref[...] = m1 / denom
    w2_ref[...] = m2 / denom

    amax = jnp.maximum(jnp.max(jnp.abs(x), axis=1, keepdims=True), 1e-8)
    sx = 127.0 / amax
    xq = jnp.clip(jnp.round(x * sx), -127.0, 127.0) / sx
    xq_ref[...] = xq.astype(jnp.bfloat16)

    a = (sel1 | sel2).astype(jnp.float32)
    inc = a
    sh = 1
    while sh < n:
        rolled = pltpu.roll(inc, sh, 0)
        riota = lax.broadcasted_iota(jnp.int32, (n, e), 0)
        inc = inc + jnp.where(riota >= sh, rolled, 0.0)
        sh *= 2
    exc = inc - a
    totals = jnp.sum(a, axis=0, keepdims=True)
    padded = jnp.ceil(totals * (1.0 / _TM)) * _TM
    ei = lax.broadcasted_iota(jnp.int32, (e, e), 0)
    ej = lax.broadcasted_iota(jnp.int32, (e, e), 1)
    m8 = (ei < ej).astype(jnp.bfloat16)
    starts = lax.dot_general(padded.astype(jnp.bfloat16), m8,
                             (((1,), (0,)), ((), ())),
                             preferred_element_type=jnp.float32)

    dst = starts + exc
    p1_ref[...] = jnp.sum(jnp.where(sel1, dst, 0.0), axis=1,
                          keepdims=True).astype(jnp.int32)
    p2_ref[...] = jnp.sum(jnp.where(sel2, dst, 0.0), axis=1,
                          keepdims=True).astype(jnp.int32)

    nt32 = tem_ref.shape[0]
    trow = (lax.broadcasted_iota(jnp.int32, (nt32, e), 0)
            .astype(jnp.float32) * float(_TM))
    tem = jnp.sum((trow >= starts).astype(jnp.int32), axis=1,
                  keepdims=True) - 1
    tem_ref[...] = tem

    fp = (totals / n) * (jnp.sum(probs, axis=0, keepdims=True) / n)
    aux_ref[...] = (_AUX_W * e) * jnp.sum(fp, keepdims=True).reshape(1, 1)


def _ffn_body(tem_ref, xs_ref, wg_ref, wu_ref, wd_ref, y_ref):
    xq = xs_ref[...]
    g = lax.dot_general(xq, wg_ref[0], (((1,), (1,)), ((), ())),
                        preferred_element_type=jnp.float32)
    u = lax.dot_general(xq, wu_ref[0], (((1,), (1,)), ((), ())),
                        preferred_element_type=jnp.float32)
    m = (g * lax.logistic(g)) * u
    amaxm = jnp.maximum(jnp.max(jnp.abs(m), axis=1, keepdims=True), 1e-8)
    sm = 127.0 / amaxm
    mq = (jnp.clip(jnp.round(m * sm), -127.0, 127.0) / sm).astype(jnp.bfloat16)
    y_ref[...] = lax.dot_general(mq, wd_ref[0], (((1,), (1,)), ((), ())),
                                 preferred_element_type=jnp.float32)


def _sc_route_gather(p1_hbm, p2_hbm, xf_hbm, xs_hbm,
                     zer_v, pos_v, ids_v, idx_v, rows_v, sem, tok_sh):
    n = p1_hbm.shape[0]
    cid = lax.axis_index("c")
    sid = lax.axis_index("s")
    wid = sid * 2 + cid
    npersc = n // 16
    rper = _RMAX // 32

    zer_v[...] = jnp.zeros_like(zer_v)
    pltpu.sync_copy(zer_v, tok_sh.at[pl.ds(sid * (_RMAX // 16), _RMAX // 16)])
    plsc.subcore_barrier()

    n0 = pl.multiple_of(sid * npersc, 8)
    for i in range(npersc // 16):
        ids_v[pl.ds(i * 16, 16)] = n0 + i * 16 + lax.iota(jnp.int32, 16)
    pltpu.sync_copy(p1_hbm.at[pl.ds(n0, npersc)], pos_v)
    pltpu.sync_copy(ids_v, tok_sh.at[pos_v], add=True)
    pltpu.sync_copy(p2_hbm.at[pl.ds(n0, npersc)], pos_v)
    pltpu.sync_copy(ids_v, tok_sh.at[pos_v], add=True)
    plsc.subcore_barrier()

    g0 = pl.multiple_of(wid * rper, 8)
    pltpu.sync_copy(tok_sh.at[pl.ds(g0, rper)], idx_v)
    pltpu.async_copy(xf_hbm.at[idx_v], rows_v, sem).wait()
    pltpu.sync_copy(rows_v, xs_hbm.at[pl.ds(g0, rper)])


def _sc_gather2(p1_hbm, p2_hbm, y_hbm, o1_hbm, o2_hbm,
                p1_v, p2_v, y1_v, y2_v, sem1, sem2):
    n = p1_hbm.shape[0]
    cid = lax.axis_index("c")
    sid = lax.axis_index("s")
    wid = sid * 2 + cid
    tper = n // 32
    t0 = pl.multiple_of(wid * tper, 8)

    pltpu.sync_copy(p1_hbm.at[pl.ds(t0, tper)], p1_v)
    pltpu.sync_copy(p2_hbm.at[pl.ds(t0, tper)], p2_v)
    cp1 = pltpu.async_copy(y_hbm.at[p1_v], y1_v, sem1)
    cp2 = pltpu.async_copy(y_hbm.at[p2_v], y2_v, sem2)
    cp1.wait()
    cp2.wait()
    pltpu.sync_copy(y1_v, o1_hbm.at[pl.ds(t0, tper)])
    pltpu.sync_copy(y2_v, o2_hbm.at[pl.ds(t0, tper)])


def _combine_body(y1_ref, y2_ref, w1_ref, w2_ref, out_ref):
    out_ref[...] = y1_ref[...] * w1_ref[...] + y2_ref[...] * w2_ref[...]


def kernel(x, Wr, Wg, Wu, Wd):
    n, d = x.shape
    e, f, _ = Wg.shape

    wgq = _quantize_weights(Wg)
    wuq = _quantize_weights(Wu)
    wdq = _quantize_weights(Wd)

    xq, p1, p2, w1, w2, tem, aux2 = pl.pallas_call(
        _router_body,
        grid=(1,),
        in_specs=[
            pl.BlockSpec((n, d), lambda i: (0, 0)),
            pl.BlockSpec((e, d), lambda i: (0, 0)),
        ],
        out_specs=[
            pl.BlockSpec((n, d), lambda i: (0, 0)),
            pl.BlockSpec((n, 1), lambda i: (0, 0)),
            pl.BlockSpec((n, 1), lambda i: (0, 0)),
            pl.BlockSpec((n, 1), lambda i: (0, 0)),
            pl.BlockSpec((n, 1), lambda i: (0, 0)),
            pl.BlockSpec((32, 1), lambda i: (0, 0)),
            pl.BlockSpec((1, 1), lambda i: (0, 0)),
        ],
        out_shape=[
            jax.ShapeDtypeStruct((n, d), jnp.bfloat16),
            jax.ShapeDtypeStruct((n, 1), jnp.int32),
            jax.ShapeDtypeStruct((n, 1), jnp.int32),
            jax.ShapeDtypeStruct((n, 1), jnp.float32),
            jax.ShapeDtypeStruct((n, 1), jnp.float32),
            jax.ShapeDtypeStruct((32, 1), jnp.int32),
            jax.ShapeDtypeStruct((1, 1), jnp.float32),
        ],
    )(x, Wr)

    p1f = p1.reshape(n)
    p2f = p2.reshape(n)
    xf = lax.bitcast_convert_type(xq.reshape(n, d // 2, 2), jnp.float32)

    mesh = plsc.VectorSubcoreMesh(core_axis_name="c", subcore_axis_name="s")
    xs = pl.kernel(
        _sc_route_gather,
        mesh=mesh,
        out_type=jax.ShapeDtypeStruct((_RMAX, d // 2), jnp.float32),
        scratch_types=[
            pltpu.VMEM((_RMAX // 16,), jnp.int32),
            pltpu.VMEM((n // 16,), jnp.int32),
            pltpu.VMEM((n // 16,), jnp.int32),
            pltpu.VMEM((_RMAX // 32,), jnp.int32),
            pltpu.VMEM((_RMAX // 32, d // 2), jnp.float32),
            pltpu.SemaphoreType.DMA,
            pltpu.VMEM_SHARED((_RMAX,), jnp.int32),
        ],
    )(p1f, p2f, xf)

    xsb = lax.bitcast_convert_type(xs, jnp.bfloat16).reshape(_RMAX, d)

    ys = pl.pallas_call(
        _ffn_body,
        grid_spec=pltpu.PrefetchScalarGridSpec(
            num_scalar_prefetch=1,
            grid=(_NTILES,),
            in_specs=[
                pl.BlockSpec((_TM, d), lambda i, tm: (i, 0)),
                pl.BlockSpec((1, f, d), lambda i, tm: (tm[i], 0, 0)),
                pl.BlockSpec((1, f, d), lambda i, tm: (tm[i], 0, 0)),
                pl.BlockSpec((1, d, f), lambda i, tm: (tm[i], 0, 0)),
            ],
            out_specs=pl.BlockSpec((_TM, d), lambda i, tm: (i, 0)),
        ),
        out_shape=jax.ShapeDtypeStruct((_RMAX, d), jnp.float32),
        compiler_params=pltpu.CompilerParams(
            vmem_limit_bytes=60 * 1024 * 1024,
        ),
    )(tem.reshape(32), xsb, wgq, wuq, wdq)

    y1, y2 = pl.kernel(
        _sc_gather2,
        mesh=mesh,
        out_type=[
            jax.ShapeDtypeStruct((n, d), jnp.float32),
            jax.ShapeDtypeStruct((n, d), jnp.float32),
        ],
        scratch_types=[
            pltpu.VMEM((n // 32,), jnp.int32),
            pltpu.VMEM((n // 32,), jnp.int32),
            pltpu.VMEM((n // 32, d), jnp.float32),
            pltpu.VMEM((n // 32, d), jnp.float32),
            pltpu.SemaphoreType.DMA,
            pltpu.SemaphoreType.DMA,
        ],
    )(p1f, p2f, ys)

    tn = 256
    out = pl.pallas_call(
        _combine_body,
        grid=(n // tn,),
        in_specs=[
            pl.BlockSpec((tn, d), lambda i: (i, 0)),
            pl.BlockSpec((tn, d), lambda i: (i, 0)),
            pl.BlockSpec((tn, 1), lambda i: (i, 0)),
            pl.BlockSpec((tn, 1), lambda i: (i, 0)),
        ],
        out_specs=pl.BlockSpec((tn, d), lambda i: (i, 0)),
        out_shape=jax.ShapeDtypeStruct((n, d), jnp.float32),
    )(y1, y2, w1, w2)

    return out, jnp.reshape(aux2, ())


# speedup vs baseline: 1.2739x; 1.2739x over previous
import functools

import jax
import jax.numpy as jnp
from jax import lax
from jax.experimental import pallas as pl
from jax.experimental.pallas import tpu as pltpu
from jax.experimental.pallas import tpu_sc as plsc

_E = 8
_AUX_W = 0.01
_TM = 256
_RMAX = 4096 + _E * _TM
_NTILES = _RMAX // _TM


def _wq_body(w_ref, q_ref):
    w = w_ref[0]
    s = jnp.maximum(jnp.mean(jnp.abs(w)), 1e-8)
    uq = jnp.clip(jnp.round(w / s), -1.0, 1.0)
    q_ref[0] = (uq * s).astype(jnp.bfloat16)


def _quantize_weights(w):
    e, r, c = w.shape
    return pl.pallas_call(
        _wq_body,
        grid=(e,),
        in_specs=[pl.BlockSpec((1, r, c), lambda i: (i, 0, 0))],
        out_specs=pl.BlockSpec((1, r, c), lambda i: (i, 0, 0)),
        out_shape=jax.ShapeDtypeStruct((e, r, c), jnp.bfloat16),
    )(w)


def _router_body(x_ref, wr_ref, xq_ref, p1_ref, p2_ref, w1_ref, w2_ref,
                 tem_ref, aux_ref):
    n = x_ref.shape[0]
    x = x_ref[...]
    logits = lax.dot_general(x.astype(jnp.bfloat16),
                             wr_ref[...].astype(jnp.bfloat16),
                             (((1,), (1,)), ((), ())),
                             preferred_element_type=jnp.float32)
    mx = jnp.max(logits, axis=1, keepdims=True)
    ex = jnp.exp(logits - mx)
    probs = ex / jnp.sum(ex, axis=1, keepdims=True)

    e = probs.shape[1]
    iota = lax.broadcasted_iota(jnp.int32, (n, e), 1)
    m1 = jnp.max(probs, axis=1, keepdims=True)
    i1 = jnp.min(jnp.where(probs == m1, iota, e), axis=1, keepdims=True)
    sel1 = iota == i1
    pm = jnp.where(sel1, -1.0, probs)
    m2 = jnp.max(pm, axis=1, keepdims=True)
    i2 = jnp.min(jnp.where(pm == m2, iota, e), axis=1, keepdims=True)
    sel2 = iota == i2
    denom = jnp.maximum(m1 + m2, 1e-9)
    w1_ref[...] = m1 / denom
    w2_ref[...] = m2 / denom

    amax = jnp.maximum(jnp.max(jnp.abs(x), axis=1, keepdims=True), 1e-8)
    sx = 127.0 / amax
    xq_ref[...] = jnp.clip(jnp.round(x * sx), -127.0, 127.0) / sx

    a = (sel1 | sel2).astype(jnp.float32)
    inc = a
    sh = 1
    while sh < n:
        rolled = pltpu.roll(inc, sh, 0)
        riota = lax.broadcasted_iota(jnp.int32, (n, e), 0)
        inc = inc + jnp.where(riota >= sh, rolled, 0.0)
        sh *= 2
    exc = inc - a
    totals = jnp.sum(a, axis=0, keepdims=True)
    padded = jnp.ceil(totals * (1.0 / _TM)) * _TM
    ei = lax.broadcasted_iota(jnp.int32, (e, e), 0)
    ej = lax.broadcasted_iota(jnp.int32, (e, e), 1)
    m8 = (ei < ej).astype(jnp.bfloat16)
    starts = lax.dot_general(padded.astype(jnp.bfloat16), m8,
                             (((1,), (0,)), ((), ())),
                             preferred_element_type=jnp.float32)

    dst = starts + exc
    p1_ref[...] = jnp.sum(jnp.where(sel1, dst, 0.0), axis=1,
                          keepdims=True).astype(jnp.int32)
    p2_ref[...] = jnp.sum(jnp.where(sel2, dst, 0.0), axis=1,
                          keepdims=True).astype(jnp.int32)

    nt32 = tem_ref.shape[0]
    trow = (lax.broadcasted_iota(jnp.int32, (nt32, e), 0)
            .astype(jnp.float32) * float(_TM))
    tem = jnp.sum((trow >= starts).astype(jnp.int32), axis=1,
                  keepdims=True) - 1
    tem_ref[...] = tem

    fp = (totals / n) * (jnp.sum(probs, axis=0, keepdims=True) / n)
    aux_ref[...] = (_AUX_W * e) * jnp.sum(fp, keepdims=True).reshape(1, 1)


def _ffn_body(tem_ref, xs_ref, rw_ref, wg_ref, wu_ref, wd_ref, y_ref):
    xq = xs_ref[...].astype(jnp.bfloat16)
    g = lax.dot_general(xq, wg_ref[0], (((1,), (1,)), ((), ())),
                        preferred_element_type=jnp.float32)
    u = lax.dot_general(xq, wu_ref[0], (((1,), (1,)), ((), ())),
                        preferred_element_type=jnp.float32)
    m = (g * lax.logistic(g)) * u
    amaxm = jnp.maximum(jnp.max(jnp.abs(m), axis=1, keepdims=True), 1e-8)
    sm = 127.0 / amaxm
    mq = (jnp.clip(jnp.round(m * sm), -127.0, 127.0) / sm).astype(jnp.bfloat16)
    d32 = lax.dot_general(mq, wd_ref[0], (((1,), (1,)), ((), ())),
                          preferred_element_type=jnp.float32)
    y_ref[...] = d32 * rw_ref[...]


def _sc_route_gather(p1_hbm, p2_hbm, w1_hbm, w2_hbm, xq_hbm, xs_hbm, rw_hbm,
                     p1_v, p2_v, w1_v, w2_v, ltok_v, lw_v, rows_v, sem):
    n = p1_hbm.shape[0]
    cid = lax.axis_index("c")
    sid = lax.axis_index("s")
    wid = sid * 2 + cid
    rper = _RMAX // 32
    g0 = pl.multiple_of(wid * rper, 8)

    pltpu.sync_copy(p1_hbm, p1_v)
    pltpu.sync_copy(p2_hbm, p2_v)
    pltpu.sync_copy(w1_hbm, w1_v)
    pltpu.sync_copy(w2_hbm, w2_v)

    for i in range(rper // 16):
        ltok_v[pl.ds(i * 16, 16)] = jnp.zeros((16,), jnp.int32)
        lw_v[pl.ds(i * 16, 16)] = jnp.zeros((16,), jnp.float32)

    def chunk(c, _):
        toks = jnp.full((16,), 0, jnp.int32) + c * 16 + lax.iota(jnp.int32, 16)
        for p_v, w_v in ((p1_v, w1_v), (p2_v, w2_v)):
            pos = p_v[pl.ds(c * 16, 16)]
            loc = pos - g0
            msk = (pos >= g0) & (pos < g0 + rper)
            plsc.store_scatter(ltok_v, [loc], toks, mask=msk)
            plsc.store_scatter(lw_v, [loc], w_v[pl.ds(c * 16, 16)], mask=msk)
        return 0

    lax.fori_loop(0, n // 16, chunk, 0)

    half = rper // 2
    cpa = pltpu.async_copy(xq_hbm.at[ltok_v.at[pl.ds(0, half)]], rows_v, sem)
    cpa.wait()
    pltpu.sync_copy(rows_v, xs_hbm.at[pl.ds(g0, half)])
    cpb = pltpu.async_copy(xq_hbm.at[ltok_v.at[pl.ds(half, half)]], rows_v, sem)
    cpb.wait()
    pltpu.sync_copy(rows_v, xs_hbm.at[pl.ds(g0 + half, half)])
    pltpu.sync_copy(lw_v, rw_hbm.at[pl.ds(g0, rper)])


def _sc_combine(p1_hbm, p2_hbm, y_hbm, out_hbm,
                p1_v, p2_v, y1_v, y2_v, sem1, sem2):
    n = p1_hbm.shape[0]
    dm = y_hbm.shape[1]
    cid = lax.axis_index("c")
    sid = lax.axis_index("s")
    wid = sid * 2 + cid
    tper = n // 32
    t0 = pl.multiple_of(wid * tper, 8)

    pltpu.sync_copy(p1_hbm.at[pl.ds(t0, tper)], p1_v)
    pltpu.sync_copy(p2_hbm.at[pl.ds(t0, tper)], p2_v)
    cp1 = pltpu.async_copy(y_hbm.at[p1_v], y1_v, sem1)
    cp2 = pltpu.async_copy(y_hbm.at[p2_v], y2_v, sem2)
    cp1.wait()
    cp2.wait()

    def row_body(r, _):
        def chunk_body(c, _):
            sl = pl.ds(c * 16, 16)
            y1_v[r, sl] = y1_v[r, sl] + y2_v[r, sl]
            return 0
        lax.fori_loop(0, dm // 16, chunk_body, 0)
        return 0

    lax.fori_loop(0, tper, row_body, 0)
    pltpu.sync_copy(y1_v, out_hbm.at[pl.ds(t0, tper)])


def kernel(x, Wr, Wg, Wu, Wd):
    n, d = x.shape
    e, f, _ = Wg.shape

    wgq = _quantize_weights(Wg)
    wuq = _quantize_weights(Wu)
    wdq = _quantize_weights(Wd)

    xq, p1, p2, w1, w2, tem, aux2 = pl.pallas_call(
        _router_body,
        grid=(1,),
        in_specs=[
            pl.BlockSpec((n, d), lambda i: (0, 0)),
            pl.BlockSpec((e, d), lambda i: (0, 0)),
        ],
        out_specs=[
            pl.BlockSpec((n, d), lambda i: (0, 0)),
            pl.BlockSpec((n, 1), lambda i: (0, 0)),
            pl.BlockSpec((n, 1), lambda i: (0, 0)),
            pl.BlockSpec((n, 1), lambda i: (0, 0)),
            pl.BlockSpec((n, 1), lambda i: (0, 0)),
            pl.BlockSpec((32, 1), lambda i: (0, 0)),
            pl.BlockSpec((1, 1), lambda i: (0, 0)),
        ],
        out_shape=[
            jax.ShapeDtypeStruct((n, d), jnp.float32),
            jax.ShapeDtypeStruct((n, 1), jnp.int32),
            jax.ShapeDtypeStruct((n, 1), jnp.int32),
            jax.ShapeDtypeStruct((n, 1), jnp.float32),
            jax.ShapeDtypeStruct((n, 1), jnp.float32),
            jax.ShapeDtypeStruct((32, 1), jnp.int32),
            jax.ShapeDtypeStruct((1, 1), jnp.float32),
        ],
    )(x, Wr)

    p1f = p1.reshape(n)
    p2f = p2.reshape(n)

    mesh = plsc.VectorSubcoreMesh(core_axis_name="c", subcore_axis_name="s")
    rper = _RMAX // 32
    xs, roww = pl.kernel(
        _sc_route_gather,
        mesh=mesh,
        out_type=[
            jax.ShapeDtypeStruct((_RMAX, d), jnp.float32),
            jax.ShapeDtypeStruct((_RMAX,), jnp.float32),
        ],
        scratch_types=[
            pltpu.VMEM((n,), jnp.int32),
            pltpu.VMEM((n,), jnp.int32),
            pltpu.VMEM((n,), jnp.float32),
            pltpu.VMEM((n,), jnp.float32),
            pltpu.VMEM((rper,), jnp.int32),
            pltpu.VMEM((rper,), jnp.float32),
            pltpu.VMEM((rper // 2, d), jnp.float32),
            pltpu.SemaphoreType.DMA,
        ],
        compiler_params=pltpu.CompilerParams(needs_layout_passes=False),
    )(p1f, p2f, w1.reshape(n), w2.reshape(n), xq)

    ys = pl.pallas_call(
        _ffn_body,
        grid_spec=pltpu.PrefetchScalarGridSpec(
            num_scalar_prefetch=1,
            grid=(_NTILES,),
            in_specs=[
                pl.BlockSpec((_TM, d), lambda i, tm: (i, 0)),
                pl.BlockSpec((_TM, 1), lambda i, tm: (i, 0)),
                pl.BlockSpec((1, f, d), lambda i, tm: (tm[i], 0, 0)),
                pl.BlockSpec((1, f, d), lambda i, tm: (tm[i], 0, 0)),
                pl.BlockSpec((1, d, f), lambda i, tm: (tm[i], 0, 0)),
            ],
            out_specs=pl.BlockSpec((_TM, d), lambda i, tm: (i, 0)),
        ),
        out_shape=jax.ShapeDtypeStruct((_RMAX, d), jnp.float32),
        compiler_params=pltpu.CompilerParams(
            vmem_limit_bytes=60 * 1024 * 1024,
        ),
    )(tem.reshape(32), xs, roww.reshape(_RMAX, 1), wgq, wuq, wdq)

    out = pl.kernel(
        _sc_combine,
        mesh=mesh,
        out_type=jax.ShapeDtypeStruct((n, d), jnp.float32),
        scratch_types=[
            pltpu.VMEM((n // 32,), jnp.int32),
            pltpu.VMEM((n // 32,), jnp.int32),
            pltpu.VMEM((n // 32, d), jnp.float32),
            pltpu.VMEM((n // 32, d), jnp.float32),
            pltpu.SemaphoreType.DMA,
            pltpu.SemaphoreType.DMA,
        ],
    )(p1f, p2f, ys)

    return out, jnp.reshape(aux2, ())
